# baseline (speedup 1.0000x reference)
import jax
import jax.numpy as jnp
from jax import lax
from jax.experimental import pallas as pl
from jax.experimental.pallas import tpu as pltpu
from jax.experimental.pallas import tpu_sc as plsc

NC = 2
NS = 16
NW = NC * NS
LANE = 128
CNTW = 16
NODE_PAD = 10240


def _edge_agg(table, src2d, dst2d, with_cnt):
    d = table.shape[1]
    nr = src2d.shape[1]
    rpt = NODE_PAD // NS
    mesh = plsc.VectorSubcoreMesh(core_axis_name="c", subcore_axis_name="s",
                                  num_cores=NC, num_subcores=NS)

    out_type = [jax.ShapeDtypeStruct((NC, NODE_PAD, d), jnp.float32)]
    scratch = [
        pltpu.VMEM((nr, LANE), jnp.int32),
        pltpu.VMEM((nr, LANE), jnp.int32),
        pltpu.VMEM((2, LANE, d), jnp.float32),
        pltpu.VMEM_SHARED((NODE_PAD, d), jnp.float32),
        pltpu.SemaphoreType.DMA,
        pltpu.SemaphoreType.DMA,
    ]
    if with_cnt:
        out_type.append(jax.ShapeDtypeStruct((NC, NODE_PAD, CNTW), jnp.float32))
        scratch += [
            pltpu.VMEM((LANE, CNTW), jnp.float32),
            pltpu.VMEM_SHARED((NODE_PAD, CNTW), jnp.float32),
        ]

    zrow = jnp.zeros((rpt, d), jnp.float32)
    zcnt = jnp.zeros((rpt, CNTW), jnp.float32)
    ones = jnp.ones((LANE, CNTW), jnp.float32)

    def body(src_hbm, dst_hbm, table_hbm, zrow_hbm, zcnt_hbm, ones_hbm,
             out_hbm, *rest):
        if with_cnt:
            (cnt_hbm, idx_s, idx_d, rows_v, acc, sem0, sem1,
             ones_v, acc_c) = rest
        else:
            idx_s, idx_d, rows_v, acc, sem0, sem1 = rest
        sems = (sem0, sem1)
        c = lax.axis_index("c")
        s = lax.axis_index("s")
        w = s * NC + c

        pltpu.sync_copy(zrow_hbm, acc.at[pl.ds(s * rpt, rpt)])
        if with_cnt:
            pltpu.sync_copy(zcnt_hbm, acc_c.at[pl.ds(s * rpt, rpt)])
            pltpu.sync_copy(ones_hbm, ones_v)
        pltpu.sync_copy(src_hbm.at[w], idx_s)
        pltpu.sync_copy(dst_hbm.at[w], idx_d)
        plsc.subcore_barrier()

        for k in range(2):
            pltpu.async_copy(table_hbm.at[idx_s.at[k]], rows_v.at[k], sems[k])

        @pl.loop(0, nr // 2)
        def _(i):
            for k in range(2):
                r = i * 2 + k
                pltpu.make_async_copy(table_hbm.at[idx_s.at[r]],
                                      rows_v.at[k], sems[k]).wait()
                pltpu.sync_copy(rows_v.at[k], acc.at[idx_d.at[r]], add=True)
                if with_cnt:
                    pltpu.sync_copy(ones_v, acc_c.at[idx_d.at[r]], add=True)

                @pl.when(r + 2 < nr)
                def _():
                    pltpu.async_copy(table_hbm.at[idx_s.at[r + 2]],
                                     rows_v.at[k], sems[k])

        plsc.subcore_barrier()
        pltpu.sync_copy(acc.at[pl.ds(s * rpt, rpt)],
                        out_hbm.at[c, pl.ds(s * rpt, rpt), :])
        if with_cnt:
            pltpu.sync_copy(acc_c.at[pl.ds(s * rpt, rpt)],
                            cnt_hbm.at[c, pl.ds(s * rpt, rpt), :])

    fn = pl.kernel(body, out_type=tuple(out_type), mesh=mesh,
                   scratch_types=tuple(scratch),
                   compiler_params=pltpu.CompilerParams(
                       use_tc_tiling_on_sc=False))
    return fn(src2d, dst2d, table, zrow, zcnt, ones)


def _dot_t(a, w):
    return lax.dot_general(a, w, (((1,), (1,)), ((), ())),
                           preferred_element_type=jnp.float32)


def _proj(x, wl, wr, b):
    n, din = x.shape
    dout = wl.shape[0]
    blk = 640
    grid = n // blk

    def body(x_ref, wl_ref, wr_ref, b_ref, xl_ref, xr_ref):
        xb = x_ref[...]
        xl_ref[...] = _dot_t(xb, wl_ref[...])
        xr_ref[...] = _dot_t(xb, wr_ref[...]) + b_ref[...]

    row = lambda i: (i, 0)
    full = lambda i: (0, 0)
    return pl.pallas_call(
        body,
        grid=(grid,),
        in_specs=[pl.BlockSpec((blk, din), row),
                  pl.BlockSpec((dout, din), full),
                  pl.BlockSpec((dout, din), full),
                  pl.BlockSpec((1, dout), full)],
        out_specs=[pl.BlockSpec((blk, dout), row),
                   pl.BlockSpec((blk, dout), row)],
        out_shape=[jax.ShapeDtypeStruct((n, dout), jnp.float32)] * 2,
    )(x, wl, wr, b.reshape(1, dout))


def _mid(p0, p1, c0, c1, xr, wl, wr, b):
    n, din = p0.shape
    dout = wl.shape[0]
    blk = 640
    grid = n // blk

    def body(p0_ref, p1_ref, c0_ref, c1_ref, xr_ref, wl_ref, wr_ref, b_ref,
             hl_ref, hr_ref, ct_ref):
        cnt = c0_ref[...] + c1_ref[...]
        ct_ref[...] = cnt
        h = (p0_ref[...] + p1_ref[...]) / jnp.maximum(cnt, 1.0) + xr_ref[...]
        h = jnp.maximum(h, 0.0)
        hl_ref[...] = _dot_t(h, wl_ref[...])
        hr_ref[...] = _dot_t(h, wr_ref[...]) + b_ref[...]

    row = lambda i: (i, 0)
    full = lambda i: (0, 0)
    return pl.pallas_call(
        body,
        grid=(grid,),
        in_specs=[pl.BlockSpec((blk, din), row),
                  pl.BlockSpec((blk, din), row),
                  pl.BlockSpec((blk, 1), row),
                  pl.BlockSpec((blk, 1), row),
                  pl.BlockSpec((blk, din), row),
                  pl.BlockSpec((dout, din), full),
                  pl.BlockSpec((dout, din), full),
                  pl.BlockSpec((1, dout), full)],
        out_specs=[pl.BlockSpec((blk, dout), row),
                   pl.BlockSpec((blk, dout), row),
                   pl.BlockSpec((blk, 1), row)],
        out_shape=[jax.ShapeDtypeStruct((n, dout), jnp.float32),
                   jax.ShapeDtypeStruct((n, dout), jnp.float32),
                   jax.ShapeDtypeStruct((n, 1), jnp.float32)],
    )(p0, p1, c0, c1, xr, wl, wr, b.reshape(1, dout))


def _final(q0, q1, ct, hr):
    n, d = q0.shape
    blk = 640
    grid = n // blk

    def body(q0_ref, q1_ref, ct_ref, hr_ref, o_ref):
        o_ref[...] = ((q0_ref[...] + q1_ref[...])
                      / jnp.maximum(ct_ref[...], 1.0) + hr_ref[...])

    row = lambda i: (i, 0)
    return pl.pallas_call(
        body,
        grid=(grid,),
        in_specs=[pl.BlockSpec((blk, d), row),
                  pl.BlockSpec((blk, d), row),
                  pl.BlockSpec((blk, 1), row),
                  pl.BlockSpec((blk, d), row)],
        out_specs=pl.BlockSpec((blk, d), row),
        out_shape=jax.ShapeDtypeStruct((n, d), jnp.float32),
    )(q0, q1, ct, hr)


def kernel(x, edge_index, W1_l, b1, W1_r, W2_l, b2, W2_r):
    n = x.shape[0]
    e = edge_index.shape[1]
    src = edge_index[0, ::-1].astype(jnp.int32)
    dst = edge_index[1, ::-1].astype(jnp.int32)

    nr = -(-e // (NW * LANE))
    nr += nr % 2
    ep = NW * nr * LANE
    src_p = jnp.concatenate(
        [src, jnp.zeros((ep - e,), jnp.int32)]).reshape(NW, nr, LANE)
    pad_dst = n + jnp.arange(ep - e, dtype=jnp.int32) % (NODE_PAD - n)
    dst_p = jnp.concatenate([dst, pad_dst]).reshape(NW, nr, LANE)
    x_pad = jnp.concatenate(
        [x, jnp.zeros((NODE_PAD - n, x.shape[1]), jnp.float32)])

    xl, xr = _proj(x_pad, W1_l, W1_r, b1)
    p, c = _edge_agg(xl, src_p, dst_p, with_cnt=True)
    hl, hr, ct = _mid(p[0], p[1], c[0, :, :1], c[1, :, :1], xr, W2_l, W2_r, b2)
    (q,) = _edge_agg(hl, src_p, dst_p, with_cnt=False)
    out = _final(q[0], q[1], ct, hr)
    return out[:n]


# speedup vs baseline: 3.6048x; 3.6048x over previous
import jax
import jax.numpy as jnp
from jax import lax
from jax.experimental import pallas as pl
from jax.experimental.pallas import tpu as pltpu
from jax.experimental.pallas import tpu_sc as plsc

NC = 2
NS = 16
NW = NC * NS
LANE = 128
CNTW = 16
NODE_PAD = 10240


def _edge_agg(table, src2d, dst2d, with_cnt):
    d = table.shape[1]
    nr = src2d.shape[1]
    rpt = NODE_PAD // NS
    mesh = plsc.VectorSubcoreMesh(core_axis_name="c", subcore_axis_name="s",
                                  num_cores=NC, num_subcores=NS)

    out_type = [jax.ShapeDtypeStruct((NC, NODE_PAD, d), jnp.float32)]
    scratch = [
        pltpu.VMEM((nr, LANE), jnp.int32),
        pltpu.VMEM((nr, LANE), jnp.int32),
        pltpu.VMEM((2, LANE, d), jnp.float32),
        pltpu.VMEM_SHARED((NODE_PAD, d), jnp.float32),
        pltpu.SemaphoreType.DMA,
        pltpu.SemaphoreType.DMA,
    ]
    if with_cnt:
        out_type.append(jax.ShapeDtypeStruct((NC, NODE_PAD, CNTW), jnp.float32))
        scratch += [
            pltpu.VMEM((LANE, CNTW), jnp.float32),
            pltpu.VMEM_SHARED((NODE_PAD, CNTW), jnp.float32),
        ]

    zrow = jnp.zeros((rpt, d), jnp.float32)
    zcnt = jnp.zeros((rpt, CNTW), jnp.float32)
    ones = jnp.ones((LANE, CNTW), jnp.float32)

    def body(src_hbm, dst_hbm, table_hbm, zrow_hbm, zcnt_hbm, ones_hbm,
             out_hbm, *rest):
        if with_cnt:
            (cnt_hbm, idx_s, idx_d, rows_v, acc, sem0, sem1,
             ones_v, acc_c) = rest
        else:
            idx_s, idx_d, rows_v, acc, sem0, sem1 = rest
        sems = (sem0, sem1)
        c = lax.axis_index("c")
        s = lax.axis_index("s")
        w = s * NC + c

        pltpu.sync_copy(zrow_hbm, acc.at[pl.ds(s * rpt, rpt)])
        if with_cnt:
            pltpu.sync_copy(zcnt_hbm, acc_c.at[pl.ds(s * rpt, rpt)])
            pltpu.sync_copy(ones_hbm, ones_v)
        pltpu.sync_copy(src_hbm.at[w], idx_s)
        pltpu.sync_copy(dst_hbm.at[w], idx_d)
        plsc.subcore_barrier()

        for k in range(2):
            pltpu.async_copy(table_hbm.at[idx_s.at[k]], rows_v.at[k], sems[k])

        @pl.loop(0, nr // 2)
        def _(i):
            for k in range(2):
                r = i * 2 + k
                pltpu.make_async_copy(table_hbm.at[idx_s.at[r]],
                                      rows_v.at[k], sems[k]).wait()
                pltpu.sync_copy(rows_v.at[k], acc.at[idx_d.at[r]], add=True)
                if with_cnt:
                    pltpu.sync_copy(ones_v, acc_c.at[idx_d.at[r]], add=True)

                @pl.when(r + 2 < nr)
                def _():
                    pltpu.async_copy(table_hbm.at[idx_s.at[r + 2]],
                                     rows_v.at[k], sems[k])

        plsc.subcore_barrier()
        pltpu.sync_copy(acc.at[pl.ds(s * rpt, rpt)],
                        out_hbm.at[c, pl.ds(s * rpt, rpt), :])
        if with_cnt:
            pltpu.sync_copy(acc_c.at[pl.ds(s * rpt, rpt)],
                            cnt_hbm.at[c, pl.ds(s * rpt, rpt), :])

    fn = pl.kernel(body, out_type=tuple(out_type), mesh=mesh,
                   scratch_types=tuple(scratch),
                   compiler_params=pltpu.CompilerParams(
                       use_tc_tiling_on_sc=False))
    return fn(src2d, dst2d, table, zrow, zcnt, ones)


def _dot_t(a, w):
    return lax.dot_general(a, w, (((1,), (1,)), ((), ())),
                           preferred_element_type=jnp.float32)


def _proj(x, wl, wr, b):
    n, din = x.shape
    dout = wl.shape[0]
    blk = 640
    grid = n // blk

    def body(x_ref, wl_ref, wr_ref, b_ref, xl_ref, xr_ref):
        xb = x_ref[...]
        xl_ref[...] = _dot_t(xb, wl_ref[...])
        xr_ref[...] = _dot_t(xb, wr_ref[...]) + b_ref[...]

    row = lambda i: (i, 0)
    full = lambda i: (0, 0)
    return pl.pallas_call(
        body,
        grid=(grid,),
        in_specs=[pl.BlockSpec((blk, din), row),
                  pl.BlockSpec((dout, din), full),
                  pl.BlockSpec((dout, din), full),
                  pl.BlockSpec((1, dout), full)],
        out_specs=[pl.BlockSpec((blk, dout), row),
                   pl.BlockSpec((blk, dout), row)],
        out_shape=[jax.ShapeDtypeStruct((n, dout), jnp.float32)] * 2,
    )(x, wl, wr, b.reshape(1, dout))


def _mid(p0, p1, c0, c1, xr, wl, wr, b):
    n, din = p0.shape
    dout = wl.shape[0]
    blk = 640
    grid = n // blk

    def body(p0_ref, p1_ref, c0_ref, c1_ref, xr_ref, wl_ref, wr_ref, b_ref,
             hl_ref, hr_ref, ct_ref):
        cnt = c0_ref[...] + c1_ref[...]
        ct_ref[...] = cnt
        h = (p0_ref[...] + p1_ref[...]) / jnp.maximum(cnt, 1.0) + xr_ref[...]
        h = jnp.maximum(h, 0.0)
        hl_ref[...] = _dot_t(h, wl_ref[...])
        hr_ref[...] = _dot_t(h, wr_ref[...]) + b_ref[...]

    row = lambda i: (i, 0)
    full = lambda i: (0, 0)
    return pl.pallas_call(
        body,
        grid=(grid,),
        in_specs=[pl.BlockSpec((blk, din), row),
                  pl.BlockSpec((blk, din), row),
                  pl.BlockSpec((blk, 1), row),
                  pl.BlockSpec((blk, 1), row),
                  pl.BlockSpec((blk, din), row),
                  pl.BlockSpec((dout, din), full),
                  pl.BlockSpec((dout, din), full),
                  pl.BlockSpec((1, dout), full)],
        out_specs=[pl.BlockSpec((blk, dout), row),
                   pl.BlockSpec((blk, dout), row),
                   pl.BlockSpec((blk, 1), row)],
        out_shape=[jax.ShapeDtypeStruct((n, dout), jnp.float32),
                   jax.ShapeDtypeStruct((n, dout), jnp.float32),
                   jax.ShapeDtypeStruct((n, 1), jnp.float32)],
    )(p0, p1, c0, c1, xr, wl, wr, b.reshape(1, dout))


def _final(q0, q1, ct, hr):
    n, d = q0.shape
    blk = 640
    grid = n // blk

    def body(q0_ref, q1_ref, ct_ref, hr_ref, o_ref):
        o_ref[...] = ((q0_ref[...] + q1_ref[...])
                      / jnp.maximum(ct_ref[...], 1.0) + hr_ref[...])

    row = lambda i: (i, 0)
    return pl.pallas_call(
        body,
        grid=(grid,),
        in_specs=[pl.BlockSpec((blk, d), row),
                  pl.BlockSpec((blk, d), row),
                  pl.BlockSpec((blk, 1), row),
                  pl.BlockSpec((blk, d), row)],
        out_specs=pl.BlockSpec((blk, d), row),
        out_shape=jax.ShapeDtypeStruct((n, d), jnp.float32),
    )(q0, q1, ct, hr)


def kernel(x, edge_index, W1_l, b1, W1_r, W2_l, b2, W2_r):
    n = x.shape[0]
    e = edge_index.shape[1]
    src = edge_index[0].astype(jnp.int32)
    dst = edge_index[1].astype(jnp.int32)

    nr = -(-e // (NW * LANE))
    nr += nr % 2
    ep = NW * nr * LANE
    pad_spread = jnp.arange(ep - e, dtype=jnp.int32) % (NODE_PAD - n)
    src_p = jnp.concatenate([src, pad_spread]).reshape(NW, nr, LANE)
    dst_p = jnp.concatenate([dst, n + pad_spread]).reshape(NW, nr, LANE)
    x_pad = jnp.concatenate(
        [x, jnp.zeros((NODE_PAD - n, x.shape[1]), jnp.float32)])

    xl, xr = _proj(x_pad, W1_l, W1_r, b1)
    p, c = _edge_agg(xl, src_p, dst_p, with_cnt=True)
    hl, hr, ct = _mid(p[0], p[1], c[0, :, :1], c[1, :, :1], xr, W2_l, W2_r, b2)
    (q,) = _edge_agg(hl, src_p, dst_p, with_cnt=False)
    out = _final(q[0], q[1], ct, hr)
    return out[:n]


# speedup vs baseline: 4.4482x; 1.2340x over previous
import jax
import jax.numpy as jnp
from jax import lax
from jax.experimental import pallas as pl
from jax.experimental.pallas import tpu as pltpu
from jax.experimental.pallas import tpu_sc as plsc

NC = 2
NS = 16
NW = NC * NS
LANE = 128
CNTW = 16
NODE_PAD = 10240


def _edge_agg(table, edges, with_cnt):
    d = table.shape[1]
    nrows = edges.shape[1]
    q, rem = divmod(nrows, NW)
    assert q % 2 == 0, "2-deep pipeline assumes an even base row count"
    rpt = NODE_PAD // NS
    mesh = plsc.VectorSubcoreMesh(core_axis_name="c", subcore_axis_name="s",
                                  num_cores=NC, num_subcores=NS)

    out_type = [jax.ShapeDtypeStruct((NC, NODE_PAD, d), jnp.float32)]
    scratch = [
        pltpu.VMEM((q + 1, LANE), jnp.int32),
        pltpu.VMEM((q + 1, LANE), jnp.int32),
        pltpu.VMEM((2, LANE, d), jnp.float32),
        pltpu.VMEM_SHARED((NODE_PAD, d), jnp.float32),
        pltpu.SemaphoreType.DMA,
        pltpu.SemaphoreType.DMA,
    ]
    if with_cnt:
        out_type.append(jax.ShapeDtypeStruct((NC, NODE_PAD, CNTW),
                                             jnp.float32))
        scratch += [
            pltpu.VMEM((LANE, CNTW), jnp.float32),
            pltpu.VMEM_SHARED((NODE_PAD, CNTW), jnp.float32),
        ]

    zrow = jnp.zeros((rpt, d), jnp.float32)
    zcnt = jnp.zeros((rpt, CNTW), jnp.float32)
    ones = jnp.ones((LANE, CNTW), jnp.float32)

    def body(edges_hbm, table_hbm, zrow_hbm, zcnt_hbm, ones_hbm,
             out_hbm, *rest):
        if with_cnt:
            (cnt_hbm, idx_s, idx_d, rows_v, acc, sem0, sem1,
             ones_v, acc_c) = rest
        else:
            idx_s, idx_d, rows_v, acc, sem0, sem1 = rest
        sems = (sem0, sem1)
        c = lax.axis_index("c")
        s = lax.axis_index("s")
        w = s * NC + c
        row0 = q * w + jnp.minimum(w, rem)
        nrw = q + (w < rem).astype(jnp.int32)

        pltpu.sync_copy(zrow_hbm, acc.at[pl.ds(s * rpt, rpt)])
        if with_cnt:
            pltpu.sync_copy(zcnt_hbm, acc_c.at[pl.ds(s * rpt, rpt)])
            pltpu.sync_copy(ones_hbm, ones_v)
        pltpu.sync_copy(edges_hbm.at[0, pl.ds(row0, q)],
                        idx_s.at[pl.ds(0, q)])
        pltpu.sync_copy(edges_hbm.at[1, pl.ds(row0, q)],
                        idx_d.at[pl.ds(0, q)])
        if rem:
            @pl.when(w < rem)
            def _():
                pltpu.sync_copy(edges_hbm.at[0, row0 + q], idx_s.at[q])
                pltpu.sync_copy(edges_hbm.at[1, row0 + q], idx_d.at[q])
        plsc.subcore_barrier()

        for k in range(2):
            pltpu.async_copy(table_hbm.at[idx_s.at[k]], rows_v.at[k], sems[k])

        def step(r, k, prefetch):
            pltpu.make_async_copy(table_hbm.at[idx_s.at[r]],
                                  rows_v.at[k], sems[k]).wait()
            pltpu.sync_copy(rows_v.at[k], acc.at[idx_d.at[r]], add=True)
            if with_cnt:
                pltpu.sync_copy(ones_v, acc_c.at[idx_d.at[r]], add=True)
            if prefetch:
                @pl.when(r + 2 < nrw)
                def _():
                    pltpu.async_copy(table_hbm.at[idx_s.at[r + 2]],
                                     rows_v.at[k], sems[k])

        @pl.loop(0, q // 2)
        def _(i):
            for k in range(2):
                step(i * 2 + k, k, True)

        if rem:
            @pl.when(w < rem)
            def _():
                step(q, q % 2, False)

        plsc.subcore_barrier()
        pltpu.sync_copy(acc.at[pl.ds(s * rpt, rpt)],
                        out_hbm.at[c, pl.ds(s * rpt, rpt), :])
        if with_cnt:
            pltpu.sync_copy(acc_c.at[pl.ds(s * rpt, rpt)],
                            cnt_hbm.at[c, pl.ds(s * rpt, rpt), :])

    fn = pl.kernel(body, out_type=tuple(out_type), mesh=mesh,
                   scratch_types=tuple(scratch),
                   compiler_params=pltpu.CompilerParams(
                       use_tc_tiling_on_sc=False))
    return fn(edges, table, zrow, zcnt, ones)


def _dot_t(a, w):
    return lax.dot_general(a, w, (((1,), (1,)), ((), ())),
                           preferred_element_type=jnp.float32)


def _proj(x, wl, wr, b):
    n, din = x.shape
    dout = wl.shape[0]
    blk = 1280
    grid = NODE_PAD // blk

    def body(x_ref, wl_ref, wr_ref, b_ref, xl_ref, xr_ref):
        xb = x_ref[...]
        xl_ref[...] = _dot_t(xb, wl_ref[...])
        xr_ref[...] = _dot_t(xb, wr_ref[...]) + b_ref[...]

    row = lambda i: (i, 0)
    full = lambda i: (0, 0)
    return pl.pallas_call(
        body,
        grid=(grid,),
        in_specs=[pl.BlockSpec((blk, din), row),
                  pl.BlockSpec((dout, din), full),
                  pl.BlockSpec((dout, din), full),
                  pl.BlockSpec((1, dout), full)],
        out_specs=[pl.BlockSpec((blk, dout), row),
                   pl.BlockSpec((blk, dout), row)],
        out_shape=[jax.ShapeDtypeStruct((NODE_PAD, dout), jnp.float32)] * 2,
    )(x, wl, wr, b.reshape(1, dout))


def _mid(p, c, xr, wl, wr, b):
    _, n, din = p.shape
    dout = wl.shape[0]
    blk = 2560
    grid = n // blk

    def body(p_ref, c_ref, xr_ref, wl_ref, wr_ref, b_ref,
             hl_ref, hr_ref, ct_ref):
        cnt = c_ref[0, :, :1] + c_ref[1, :, :1]
        ct_ref[...] = cnt
        h = (p_ref[0] + p_ref[1]) / jnp.maximum(cnt, 1.0) + xr_ref[...]
        h = jnp.maximum(h, 0.0)
        hl_ref[...] = _dot_t(h, wl_ref[...])
        hr_ref[...] = _dot_t(h, wr_ref[...]) + b_ref[...]

    row3 = lambda i: (0, i, 0)
    row = lambda i: (i, 0)
    full = lambda i: (0, 0)
    return pl.pallas_call(
        body,
        grid=(grid,),
        in_specs=[pl.BlockSpec((2, blk, din), row3),
                  pl.BlockSpec((2, blk, CNTW), row3),
                  pl.BlockSpec((blk, din), row),
                  pl.BlockSpec((dout, din), full),
                  pl.BlockSpec((dout, din), full),
                  pl.BlockSpec((1, dout), full)],
        out_specs=[pl.BlockSpec((blk, dout), row),
                   pl.BlockSpec((blk, dout), row),
                   pl.BlockSpec((blk, 1), row)],
        out_shape=[jax.ShapeDtypeStruct((n, dout), jnp.float32),
                   jax.ShapeDtypeStruct((n, dout), jnp.float32),
                   jax.ShapeDtypeStruct((n, 1), jnp.float32)],
    )(p, c, xr, wl, wr, b.reshape(1, dout))


def _final(qp, ct, hr, n_out):
    _, _, d = qp.shape
    blk = 2000
    grid = n_out // blk

    def body(q_ref, ct_ref, hr_ref, o_ref):
        o_ref[...] = ((q_ref[0] + q_ref[1])
                      / jnp.maximum(ct_ref[...], 1.0) + hr_ref[...])

    row3 = lambda i: (0, i, 0)
    row = lambda i: (i, 0)
    return pl.pallas_call(
        body,
        grid=(grid,),
        in_specs=[pl.BlockSpec((2, blk, d), row3),
                  pl.BlockSpec((blk, 1), row),
                  pl.BlockSpec((blk, d), row)],
        out_specs=pl.BlockSpec((blk, d), row),
        out_shape=jax.ShapeDtypeStruct((n_out, d), jnp.float32),
    )(qp, ct, hr)


def kernel(x, edge_index, W1_l, b1, W1_r, W2_l, b2, W2_r):
    n = x.shape[0]
    e = edge_index.shape[1]
    assert e % LANE == 0
    edges = edge_index.astype(jnp.int32).reshape(2, e // LANE, LANE)

    xl, xr = _proj(x, W1_l, W1_r, b1)
    p, c = _edge_agg(xl, edges, with_cnt=True)
    hl, hr, ct = _mid(p, c, xr, W2_l, W2_r, b2)
    (qp,) = _edge_agg(hl, edges, with_cnt=False)
    return _final(qp, ct, hr, n)


# speedup vs baseline: 4.4740x; 1.0058x over previous
import jax
import jax.numpy as jnp
from jax import lax
from jax.experimental import pallas as pl
from jax.experimental.pallas import tpu as pltpu
from jax.experimental.pallas import tpu_sc as plsc

NC = 2
NS = 16
NW = NC * NS
LANE = 128
CNTW = 16
NODE_PAD = 10240


def _edge_agg(table, edges, with_cnt):
    d = table.shape[1]
    nrows = edges.shape[1]
    q, rem = divmod(nrows, NW)
    assert q % 2 == 0, "2-deep pipeline assumes an even base row count"
    rpt = NODE_PAD // NS
    mesh = plsc.VectorSubcoreMesh(core_axis_name="c", subcore_axis_name="s",
                                  num_cores=NC, num_subcores=NS)

    out_type = [jax.ShapeDtypeStruct((NC, NODE_PAD, d), jnp.float32)]
    scratch = [
        pltpu.VMEM((q + 1, LANE), jnp.int32),
        pltpu.VMEM((q + 1, LANE), jnp.int32),
        pltpu.VMEM((2, LANE, d), jnp.float32),
        pltpu.VMEM_SHARED((NODE_PAD, d), jnp.float32),
        pltpu.SemaphoreType.DMA,
        pltpu.SemaphoreType.DMA,
    ]
    if with_cnt:
        out_type.append(jax.ShapeDtypeStruct((NC, NODE_PAD, CNTW),
                                             jnp.float32))
        scratch += [
            pltpu.VMEM((LANE, CNTW), jnp.float32),
            pltpu.VMEM_SHARED((NODE_PAD, CNTW), jnp.float32),
            pltpu.SemaphoreType.DMA,
        ]

    zrow = jnp.zeros((rpt, d), jnp.float32)
    zcnt = jnp.zeros((rpt, CNTW), jnp.float32)
    ones = jnp.ones((LANE, CNTW), jnp.float32)

    def body(edges_hbm, table_hbm, zrow_hbm, zcnt_hbm, ones_hbm,
             out_hbm, *rest):
        if with_cnt:
            (cnt_hbm, idx_s, idx_d, rows_v, acc, sem0, sem1,
             ones_v, acc_c, sem_c) = rest
        else:
            idx_s, idx_d, rows_v, acc, sem0, sem1 = rest
        sems = (sem0, sem1)
        c = lax.axis_index("c")
        s = lax.axis_index("s")
        w = s * NC + c
        row0 = q * w + jnp.minimum(w, rem)
        nrw = q + (w < rem).astype(jnp.int32)

        pltpu.sync_copy(zrow_hbm, acc.at[pl.ds(s * rpt, rpt)])
        if with_cnt:
            pltpu.sync_copy(zcnt_hbm, acc_c.at[pl.ds(s * rpt, rpt)])
            pltpu.sync_copy(ones_hbm, ones_v)
        pltpu.sync_copy(edges_hbm.at[0, pl.ds(row0, q)],
                        idx_s.at[pl.ds(0, q)])
        pltpu.sync_copy(edges_hbm.at[1, pl.ds(row0, q)],
                        idx_d.at[pl.ds(0, q)])
        if rem:
            @pl.when(w < rem)
            def _():
                pltpu.sync_copy(edges_hbm.at[0, row0 + q], idx_s.at[q])
                pltpu.sync_copy(edges_hbm.at[1, row0 + q], idx_d.at[q])
        plsc.subcore_barrier()

        for k in range(2):
            pltpu.async_copy(table_hbm.at[idx_s.at[k]], rows_v.at[k], sems[k])

        def step(r, k, prefetch):
            pltpu.make_async_copy(table_hbm.at[idx_s.at[r]],
                                  rows_v.at[k], sems[k]).wait()
            if with_cnt:
                pltpu.async_copy(ones_v, acc_c.at[idx_d.at[r]], sem_c,
                                 add=True)
            pltpu.sync_copy(rows_v.at[k], acc.at[idx_d.at[r]], add=True)
            if prefetch:
                @pl.when(r + 2 < nrw)
                def _():
                    pltpu.async_copy(table_hbm.at[idx_s.at[r + 2]],
                                     rows_v.at[k], sems[k])

        @pl.loop(0, q // 2)
        def _(i):
            for k in range(2):
                step(i * 2 + k, k, True)

        if rem:
            @pl.when(w < rem)
            def _():
                step(q, q % 2, False)

        if with_cnt:
            @pl.loop(0, nrw)
            def _(r):
                pltpu.make_async_copy(ones_v, acc_c.at[idx_d.at[0]],
                                      sem_c).wait()

        plsc.subcore_barrier()
        pltpu.sync_copy(acc.at[pl.ds(s * rpt, rpt)],
                        out_hbm.at[c, pl.ds(s * rpt, rpt), :])
        if with_cnt:
            pltpu.sync_copy(acc_c.at[pl.ds(s * rpt, rpt)],
                            cnt_hbm.at[c, pl.ds(s * rpt, rpt), :])

    fn = pl.kernel(body, out_type=tuple(out_type), mesh=mesh,
                   scratch_types=tuple(scratch),
                   compiler_params=pltpu.CompilerParams(
                       use_tc_tiling_on_sc=False))
    return fn(edges, table, zrow, zcnt, ones)


def _dot_t(a, w):
    return lax.dot_general(a, w, (((1,), (1,)), ((), ())),
                           preferred_element_type=jnp.float32)


def _proj(x, wl, wr, b):
    n, din = x.shape
    dout = wl.shape[0]
    blk = 1280
    grid = NODE_PAD // blk

    def body(x_ref, wl_ref, wr_ref, b_ref, xl_ref, xr_ref):
        xb = x_ref[...]
        xl_ref[...] = _dot_t(xb, wl_ref[...])
        xr_ref[...] = _dot_t(xb, wr_ref[...]) + b_ref[...]

    row = lambda i: (i, 0)
    full = lambda i: (0, 0)
    return pl.pallas_call(
        body,
        grid=(grid,),
        in_specs=[pl.BlockSpec((blk, din), row),
                  pl.BlockSpec((dout, din), full),
                  pl.BlockSpec((dout, din), full),
                  pl.BlockSpec((1, dout), full)],
        out_specs=[pl.BlockSpec((blk, dout), row),
                   pl.BlockSpec((blk, dout), row)],
        out_shape=[jax.ShapeDtypeStruct((NODE_PAD, dout), jnp.float32)] * 2,
    )(x, wl, wr, b.reshape(1, dout))


def _mid(p, c, xr, wl, wr, b):
    _, n, din = p.shape
    dout = wl.shape[0]
    blk = 2560
    grid = n // blk

    def body(p_ref, c_ref, xr_ref, wl_ref, wr_ref, b_ref,
             hl_ref, hr_ref, ct_ref):
        cnt = c_ref[0, :, :1] + c_ref[1, :, :1]
        ct_ref[...] = cnt
        h = (p_ref[0] + p_ref[1]) / jnp.maximum(cnt, 1.0) + xr_ref[...]
        h = jnp.maximum(h, 0.0)
        hl_ref[...] = _dot_t(h, wl_ref[...])
        hr_ref[...] = _dot_t(h, wr_ref[...]) + b_ref[...]

    row3 = lambda i: (0, i, 0)
    row = lambda i: (i, 0)
    full = lambda i: (0, 0)
    return pl.pallas_call(
        body,
        grid=(grid,),
        in_specs=[pl.BlockSpec((2, blk, din), row3),
                  pl.BlockSpec((2, blk, CNTW), row3),
                  pl.BlockSpec((blk, din), row),
                  pl.BlockSpec((dout, din), full),
                  pl.BlockSpec((dout, din), full),
                  pl.BlockSpec((1, dout), full)],
        out_specs=[pl.BlockSpec((blk, dout), row),
                   pl.BlockSpec((blk, dout), row),
                   pl.BlockSpec((blk, 1), row)],
        out_shape=[jax.ShapeDtypeStruct((n, dout), jnp.float32),
                   jax.ShapeDtypeStruct((n, dout), jnp.float32),
                   jax.ShapeDtypeStruct((n, 1), jnp.float32)],
    )(p, c, xr, wl, wr, b.reshape(1, dout))


def _final(qp, ct, hr, n_out):
    _, _, d = qp.shape
    blk = 2000
    grid = n_out // blk

    def body(q_ref, ct_ref, hr_ref, o_ref):
        o_ref[...] = ((q_ref[0] + q_ref[1])
                      / jnp.maximum(ct_ref[...], 1.0) + hr_ref[...])

    row3 = lambda i: (0, i, 0)
    row = lambda i: (i, 0)
    return pl.pallas_call(
        body,
        grid=(grid,),
        in_specs=[pl.BlockSpec((2, blk, d), row3),
                  pl.BlockSpec((blk, 1), row),
                  pl.BlockSpec((blk, d), row)],
        out_specs=pl.BlockSpec((blk, d), row),
        out_shape=jax.ShapeDtypeStruct((n_out, d), jnp.float32),
    )(qp, ct, hr)


def kernel(x, edge_index, W1_l, b1, W1_r, W2_l, b2, W2_r):
    n = x.shape[0]
    e = edge_index.shape[1]
    assert e % LANE == 0
    edges = edge_index.astype(jnp.int32).reshape(2, e // LANE, LANE)

    xl, xr = _proj(x, W1_l, W1_r, b1)
    p, c = _edge_agg(xl, edges, with_cnt=True)
    hl, hr, ct = _mid(p, c, xr, W2_l, W2_r, b2)
    (qp,) = _edge_agg(hl, edges, with_cnt=False)
    return _final(qp, ct, hr, n)
